# baseline (speedup 1.0000x reference)
import functools

import jax
import jax.numpy as jnp
from jax import lax
from jax.experimental import pallas as pl
from jax.experimental.pallas import tpu as pltpu
from jax.experimental.pallas import tpu_sc as plsc

_F32 = jnp.float32


def _sc_sparse(operations, items, related_items, materials, resources,
               res_ei, mat_ei, prec_ei):
    n_op, dim = operations.shape
    assert dim == 128
    acc_rows = ((n_op + 1 + 255) // 256) * 256
    zr = acc_rows // 16
    zh = zr // 16

    def col4(t):
        return t.reshape(4 * t.shape[0], 32)

    def prep(dst, src, v):
        e = dst.shape[0]
        e_pad = ((e + 8191) // 8192) * 8192
        dstp = jnp.concatenate(
            [dst, jnp.full((e_pad - e,), n_op, jnp.int32)])
        srcp = jnp.concatenate([src, jnp.zeros((e_pad - e,), jnp.int32)])
        src4 = srcp[None, :] * 4 + jnp.arange(4, dtype=jnp.int32)[:, None]
        return dstp.reshape(-1, 128), src4.reshape(4, -1, 128), e_pad

    ops4 = col4(operations)
    res4 = col4(resources)
    mat4 = col4(materials)
    pd0, ps0, e_prec = prep(prec_ei[0], prec_ei[1], n_op)
    pd1, ps1, _ = prep(prec_ei[1], prec_ei[0], n_op)
    rd, rs, e_res = prep(res_ei[0], res_ei[1], resources.shape[0])
    md, ms, e_mat = prep(mat_ei[0], mat_ei[1], materials.shape[0])

    n_items = related_items.shape[0]
    ip = ((n_items + 1023) // 1024) * 1024
    iidx = jnp.concatenate(
        [related_items, jnp.zeros((ip - n_items,), jnp.int32)])

    mesh = plsc.VectorSubcoreMesh(core_axis_name="c", subcore_axis_name="s")
    agg_t = jax.ShapeDtypeStruct((4, acc_rows, 32), _F32)

    @functools.partial(
        pl.kernel,
        out_type=[agg_t, agg_t, agg_t, agg_t,
                  jax.ShapeDtypeStruct((ip, dim), _F32)],
        mesh=mesh,
        compiler_params=pltpu.CompilerParams(use_tc_tiling_on_sc=False),
        scratch_types=[
            pltpu.VMEM_SHARED((acc_rows, 32), _F32),
            pltpu.VMEM((zh, 32), _F32),
            pltpu.VMEM((4, 128), jnp.int32),
            pltpu.VMEM((4, 128), jnp.int32),
            pltpu.VMEM((4, 128, 32), _F32),
            pltpu.VMEM((32, dim), _F32),
            pltpu.VMEM((32,), jnp.int32),
            pltpu.SemaphoreType.DMA,
        ] + [pltpu.SemaphoreType.DMA] * 13,
    )
    def sc_kernel(ops4_h, pd0_h, ps0_h, pd1_h, ps1_h, res4_h, rd_h, rs_h,
                  mat4_h, md_h, ms_h, itab_h, iidx_h,
                  o_pred, o_succ, o_res, o_mat, o_item,
                  acc, zbuf, dstb, srcb, rows, gbuf, gidx, sem, *sems):
        c = lax.axis_index("c")
        s = lax.axis_index("s")

        w = s * 2 + c
        nbi = ip // 32 // 32

        @pl.loop(0, nbi)
        def _(j):
            off = (j * 32 + w) * 32
            pltpu.sync_copy(iidx_h.at[pl.ds(off, 32)], gidx)
            pltpu.async_copy(itab_h.at[gidx], gbuf, sem).wait()
            pltpu.sync_copy(gbuf, o_item.at[pl.ds(off, 32)])

        @pl.loop(0, zh)
        def _(i):
            zbuf[i, pl.ds(0, 16)] = jnp.zeros((16,), _F32)
            zbuf[i, pl.ds(16, 16)] = jnp.zeros((16,), _F32)

        isem_d, isem_s = sems[0:2], sems[2:4]
        gsems, ssems, zsem = sems[4:8], sems[8:12], sems[12]

        def run_agg(tab4, d_hbm, s4_hbm, o_hbm, e_pad):
            nsb = e_pad // 8192
            for p in range(2):
                q = 2 * c + p
                for zi in range(16):
                    pltpu.sync_copy(zbuf, acc.at[pl.ds(s * zr + zi * zh, zh)])
                plsc.subcore_barrier()

                @pl.loop(0, nsb)
                def _(j):
                    rb = (j * 16 + s) * 4
                    dd = pltpu.async_copy(d_hbm.at[pl.ds(rb, 4)], dstb,
                                          isem_d[0])
                    sd = pltpu.async_copy(s4_hbm.at[q, pl.ds(rb, 4)], srcb,
                                          isem_s[0])
                    dd.wait()
                    sd.wait()
                    gd = [pltpu.async_copy(tab4.at[srcb.at[u]], rows.at[u],
                                           gsems[u]) for u in range(4)]
                    st = []
                    for u in range(4):
                        gd[u].wait()
                        st.append(pltpu.async_copy(rows.at[u],
                                                   acc.at[dstb.at[u]],
                                                   ssems[u], add=True))
                    for u in range(4):
                        st[u].wait()
                plsc.subcore_barrier()
                pltpu.sync_copy(acc.at[pl.ds(s * zr, zr)],
                                o_hbm.at[q, pl.ds(s * zr, zr)])
                plsc.subcore_barrier()

        run_agg(ops4_h, pd0_h, ps0_h, o_pred, e_prec)
        run_agg(ops4_h, pd1_h, ps1_h, o_succ, e_prec)
        run_agg(res4_h, rd_h, rs_h, o_res, e_res)
        run_agg(mat4_h, md_h, ms_h, o_mat, e_mat)

    return sc_kernel(ops4, pd0, ps0, pd1, ps1, res4, rd, rs, mat4, md, ms,
                     items, iidx)


def _elu(x):
    return jnp.where(x > 0, x, jnp.exp(jnp.minimum(x, 0.0)) - 1.0)


def _tc_body(ops_ref, item_ref, pred_ref, succ_ref, res_ref, mat_ref,
             wa1, wa2, wa3, ba1, ba2, ba3,
             wc1, wc2, wc3, bc1, bc2, bc3, o_ref):
    def mlp_tail(j, h):
        h = _elu(h)
        h = _elu(jnp.dot(h, wa2[j], preferred_element_type=_F32) + ba2[j])
        return jnp.dot(h, wa3[j], preferred_element_type=_F32) + ba3[j]

    def mlp(j, x):
        return mlp_tail(j, jnp.dot(x, wa1[j], preferred_element_type=_F32)
                        + ba1[j])

    def mlp_parts(j, ref):
        h = ba1[j]
        for qq in range(4):
            h = h + jnp.dot(ref[qq], wa1[j, pl.ds(32 * qq, 32), :],
                            preferred_element_type=_F32)
        return mlp_tail(j, h)

    e_self = mlp(0, ops_ref[...])
    e_item = mlp(1, item_ref[...])
    e_pred = mlp_parts(2, pred_ref)
    e_succ = mlp_parts(3, succ_ref)
    e_res = mlp_parts(4, res_ref)
    e_mat = mlp_parts(5, mat_ref)

    acc = bc1[...]
    for j, e in enumerate((e_pred, e_succ, e_res, e_mat, e_item, e_self)):
        acc = acc + jnp.dot(e, wc1[pl.ds(128 * j, 128), :],
                            preferred_element_type=_F32)
    h = _elu(acc)
    h = _elu(jnp.dot(h, wc2[...], preferred_element_type=_F32) + bc2[...])
    o_ref[...] = jnp.dot(h, wc3[...], preferred_element_type=_F32) + bc3[...]


def _fused_mlps(ops, item_g, agg_pred, agg_succ, agg_res, agg_mat, params):
    n_op, dim = ops.shape
    bm = 2000 if n_op % 2000 == 0 else n_op

    names = ("self", "items", "predecessors", "successors", "resources",
             "materials")
    wa1 = jnp.stack([params[k]["W1"] for k in names])
    wa2 = jnp.stack([params[k]["W2"] for k in names])
    wa3 = jnp.stack([params[k]["W3"] for k in names])
    ba1 = jnp.stack([params[k]["b1"][None, :] for k in names])
    ba2 = jnp.stack([params[k]["b2"][None, :] for k in names])
    ba3 = jnp.stack([params[k]["b3"][None, :] for k in names])
    pc = params["combined"]

    row_spec = pl.BlockSpec((bm, dim), lambda i: (i, 0))
    agg_spec = pl.BlockSpec((4, bm, 32), lambda i: (0, i, 0))
    full = lambda a: pl.BlockSpec(a.shape, lambda i: tuple(0 for _ in a.shape))
    weights = [wa1, wa2, wa3, ba1, ba2, ba3,
               pc["W1"], pc["W2"], pc["W3"],
               pc["b1"][None, :], pc["b2"][None, :], pc["b3"][None, :]]

    return pl.pallas_call(
        _tc_body,
        grid=(n_op // bm,),
        in_specs=[row_spec] * 2 + [agg_spec] * 4 + [full(w) for w in weights],
        out_specs=row_spec,
        out_shape=jax.ShapeDtypeStruct((n_op, dim), jnp.float32),
    )(ops, item_g, agg_pred, agg_succ, agg_res, agg_mat, *weights)


def kernel(operations, items, related_items, materials, resources,
           need_for_resources_edge_index, need_for_materials_edge_index,
           precedences_edge_index, params):
    agg_pred, agg_succ, agg_res, agg_mat, item_g = _sc_sparse(
        operations, items, related_items, materials, resources,
        need_for_resources_edge_index, need_for_materials_edge_index,
        precedences_edge_index)
    return _fused_mlps(operations, item_g, agg_pred, agg_succ, agg_res,
                       agg_mat, params)


# speedup vs baseline: 1.1263x; 1.1263x over previous
import functools

import jax
import jax.numpy as jnp
from jax import lax
from jax.experimental import pallas as pl
from jax.experimental.pallas import tpu as pltpu
from jax.experimental.pallas import tpu_sc as plsc

_F32 = jnp.float32


def _sc_stage(jobs, gather, n_op, dim):
    acc_rows = ((n_op + 1 + 255) // 256) * 256
    zr = acc_rows // 16
    zh = zr // 16

    mesh = plsc.VectorSubcoreMesh(core_axis_name="c", subcore_axis_name="s")
    agg_t = jax.ShapeDtypeStruct((4, acc_rows, 32), _F32)
    out_type = [agg_t] * len(jobs)
    args = []
    for tab4, d2, s42, _ in jobs:
        args += [tab4, d2, s42]
    if gather is not None:
        itab, iidx, ip = gather
        out_type.append(jax.ShapeDtypeStruct((ip, dim), _F32))
        args += [itab, iidx]

    @functools.partial(
        pl.kernel,
        out_type=out_type,
        mesh=mesh,
        compiler_params=pltpu.CompilerParams(use_tc_tiling_on_sc=False),
        scratch_types=[
            pltpu.VMEM_SHARED((acc_rows, 32), _F32),
            pltpu.VMEM((zh, 32), _F32),
            pltpu.VMEM((4, 128), jnp.int32),
            pltpu.VMEM((4, 128), jnp.int32),
            pltpu.VMEM((4, 128, 32), _F32),
            pltpu.VMEM((32, dim), _F32),
            pltpu.VMEM((32,), jnp.int32),
            pltpu.SemaphoreType.DMA,
        ] + [pltpu.SemaphoreType.DMA] * 13,
    )
    def sc_kernel(*refs):
        n_in = len(args)
        ins, outs = refs[:n_in], refs[n_in:n_in + len(out_type)]
        (acc, zbuf, dstb, srcb, rows, gbuf, gidx, sem,
         *sems) = refs[n_in + len(out_type):]
        c = lax.axis_index("c")
        s = lax.axis_index("s")

        if gather is not None:
            itab_h, iidx_h = ins[-2], ins[-1]
            o_item = outs[-1]
            w = s * 2 + c
            nbi = ip // 32 // 32

            @pl.loop(0, nbi)
            def _(j):
                off = (j * 32 + w) * 32
                pltpu.sync_copy(iidx_h.at[pl.ds(off, 32)], gidx)
                pltpu.async_copy(itab_h.at[gidx], gbuf, sem).wait()
                pltpu.sync_copy(gbuf, o_item.at[pl.ds(off, 32)])

        @pl.loop(0, zh)
        def _(i):
            zbuf[i, pl.ds(0, 16)] = jnp.zeros((16,), _F32)
            zbuf[i, pl.ds(16, 16)] = jnp.zeros((16,), _F32)

        isem_d, isem_s = sems[0:2], sems[2:4]
        gsems, ssems = sems[4:8], sems[8:12]

        def run_agg(tab4, d_hbm, s4_hbm, o_hbm, e_pad):
            nsb = e_pad // 8192
            for p in range(2):
                q = 2 * c + p
                for zi in range(16):
                    pltpu.sync_copy(zbuf, acc.at[pl.ds(s * zr + zi * zh, zh)])
                plsc.subcore_barrier()

                @pl.loop(0, nsb)
                def _(j):
                    rb = (j * 16 + s) * 4
                    dd = pltpu.async_copy(d_hbm.at[pl.ds(rb, 4)], dstb,
                                          isem_d[0])
                    sd = pltpu.async_copy(s4_hbm.at[q, pl.ds(rb, 4)], srcb,
                                          isem_s[0])
                    dd.wait()
                    sd.wait()
                    gd = [pltpu.async_copy(tab4.at[srcb.at[u]], rows.at[u],
                                           gsems[u]) for u in range(4)]
                    st = []
                    for u in range(4):
                        gd[u].wait()
                        st.append(pltpu.async_copy(rows.at[u],
                                                   acc.at[dstb.at[u]],
                                                   ssems[u], add=True))
                    for u in range(4):
                        st[u].wait()
                plsc.subcore_barrier()
                pltpu.sync_copy(acc.at[pl.ds(s * zr, zr)],
                                o_hbm.at[q, pl.ds(s * zr, zr)])
                plsc.subcore_barrier()

        for ji, (_, _, _, e_pad) in enumerate(jobs):
            run_agg(ins[3 * ji], ins[3 * ji + 1], ins[3 * ji + 2],
                    outs[ji], e_pad)

    return sc_kernel(*args)


def _col4(t):
    v = t.shape[0]
    return t.reshape(v, 4, 32).transpose(1, 0, 2).reshape(4 * v, 32)


def _prep(dst, src, v, n_op):
    e = dst.shape[0]
    e_pad = ((e + 8191) // 8192) * 8192
    dstp = jnp.concatenate([dst, jnp.full((e_pad - e,), n_op, jnp.int32)])
    srcp = jnp.concatenate([src, jnp.zeros((e_pad - e,), jnp.int32)])
    src4 = srcp[None, :] + (jnp.arange(4, dtype=jnp.int32) * v)[:, None]
    return dstp.reshape(-1, 128), src4.reshape(4, -1, 128), e_pad


def _elu(x):
    return jnp.where(x > 0, x, jnp.exp(jnp.minimum(x, 0.0)) - 1.0)


def _mlp_tail(j, h, wa2, wa3, ba2, ba3):
    h = _elu(h)
    h = _elu(jnp.dot(h, wa2[j], preferred_element_type=_F32) + ba2[j])
    return jnp.dot(h, wa3[j], preferred_element_type=_F32) + ba3[j]


def _mlp(j, x, wa1, wa2, wa3, ba1, ba2, ba3):
    return _mlp_tail(j, jnp.dot(x, wa1[j], preferred_element_type=_F32)
                     + ba1[j], wa2, wa3, ba2, ba3)


def _mlp_parts(j, ref, wa1, wa2, wa3, ba1, ba2, ba3):
    h = ba1[j]
    for qq in range(4):
        h = h + jnp.dot(ref[qq], wa1[j, pl.ds(32 * qq, 32), :],
                        preferred_element_type=_F32)
    return _mlp_tail(j, h, wa2, wa3, ba2, ba3)


def _tc_a_body(ops_ref, item_ref, pred_ref, succ_ref,
               wa1, wa2, wa3, ba1, ba2, ba3, wc1, bc1, o_ref):
    w = (wa1, wa2, wa3, ba1, ba2, ba3)
    e_self = _mlp(0, ops_ref[...], *w)
    e_item = _mlp(1, item_ref[...], *w)
    e_pred = _mlp_parts(2, pred_ref, *w)
    e_succ = _mlp_parts(3, succ_ref, *w)
    acc = bc1[...]
    for j, e in ((0, e_pred), (1, e_succ), (4, e_item), (5, e_self)):
        acc = acc + jnp.dot(e, wc1[pl.ds(128 * j, 128), :],
                            preferred_element_type=_F32)
    o_ref[...] = acc


def _tc_b_body(part_ref, res_ref, mat_ref,
               wa1, wa2, wa3, ba1, ba2, ba3,
               wc1, wc2, wc3, bc2, bc3, o_ref):
    w = (wa1, wa2, wa3, ba1, ba2, ba3)
    e_res = _mlp_parts(4, res_ref, *w)
    e_mat = _mlp_parts(5, mat_ref, *w)
    acc = part_ref[...]
    for j, e in ((2, e_res), (3, e_mat)):
        acc = acc + jnp.dot(e, wc1[pl.ds(128 * j, 128), :],
                            preferred_element_type=_F32)
    h = _elu(acc)
    h = _elu(jnp.dot(h, wc2[...], preferred_element_type=_F32) + bc2[...])
    o_ref[...] = jnp.dot(h, wc3[...], preferred_element_type=_F32) + bc3[...]


def _tc_call(body, data, weights, n_op, dim, n_agg, n_rows):
    bm = 2000 if n_op % 2000 == 0 else n_op
    row_spec = pl.BlockSpec((bm, dim), lambda i: (i, 0))
    agg_spec = pl.BlockSpec((4, bm, 32), lambda i: (0, i, 0))
    full = lambda a: pl.BlockSpec(a.shape, lambda i: tuple(0 for _ in a.shape))
    return pl.pallas_call(
        body,
        grid=(n_op // bm,),
        in_specs=[row_spec] * n_rows + [agg_spec] * n_agg
                 + [full(w) for w in weights],
        out_specs=row_spec,
        out_shape=jax.ShapeDtypeStruct((n_op, dim), jnp.float32),
    )(*data, *weights)


def kernel(operations, items, related_items, materials, resources,
           need_for_resources_edge_index, need_for_materials_edge_index,
           precedences_edge_index, params):
    n_op, dim = operations.shape
    res_ei = need_for_resources_edge_index
    mat_ei = need_for_materials_edge_index
    prec_ei = precedences_edge_index

    ops4 = _col4(operations)
    res4 = _col4(resources)
    mat4 = _col4(materials)
    pd0, ps0, e_prec = _prep(prec_ei[0], prec_ei[1], n_op, n_op)
    pd1, ps1, _ = _prep(prec_ei[1], prec_ei[0], n_op, n_op)
    rd, rs, e_res = _prep(res_ei[0], res_ei[1], resources.shape[0], n_op)
    md, ms, e_mat = _prep(mat_ei[0], mat_ei[1], materials.shape[0], n_op)

    n_items = related_items.shape[0]
    ip = ((n_items + 1023) // 1024) * 1024
    iidx = jnp.concatenate(
        [related_items, jnp.zeros((ip - n_items,), jnp.int32)])

    agg_pred, agg_succ, item_g = _sc_stage(
        [(ops4, pd0, ps0, e_prec), (ops4, pd1, ps1, e_prec)],
        (items, iidx, ip), n_op, dim)
    agg_res, agg_mat = _sc_stage(
        [(res4, rd, rs, e_res), (mat4, md, ms, e_mat)], None, n_op, dim)

    names = ("self", "items", "predecessors", "successors", "resources",
             "materials")
    wa1 = jnp.stack([params[k]["W1"] for k in names])
    wa2 = jnp.stack([params[k]["W2"] for k in names])
    wa3 = jnp.stack([params[k]["W3"] for k in names])
    ba1 = jnp.stack([params[k]["b1"][None, :] for k in names])
    ba2 = jnp.stack([params[k]["b2"][None, :] for k in names])
    ba3 = jnp.stack([params[k]["b3"][None, :] for k in names])
    pc = params["combined"]
    wa = [wa1, wa2, wa3, ba1, ba2, ba3]

    part = _tc_call(_tc_a_body, [operations, item_g, agg_pred, agg_succ],
                    wa + [pc["W1"], pc["b1"][None, :]], n_op, dim,
                    n_agg=2, n_rows=2)
    return _tc_call(_tc_b_body, [part, agg_res, agg_mat],
                    wa + [pc["W1"], pc["W2"], pc["W3"],
                          pc["b2"][None, :], pc["b3"][None, :]],
                    n_op, dim, n_agg=2, n_rows=1)
